# Initial kernel scaffold; baseline (speedup 1.0000x reference)
import jax
import jax.numpy as jnp
from jax.experimental import pallas as pl


def kernel(x, edge_index, batch, conv1_Wrel, conv1_brel, conv1_Wroot, conv2_Wrel, conv2_brel, conv2_Wroot, conv3_Wrel, conv3_brel, conv3_Wroot, pool1_w, pool2_w, lin1_W, lin1_b, lin2_W, lin2_b):
    raise NotImplementedError("write your pallas kernel here")


# speedup vs baseline: 1.3204x; 1.3204x over previous
import math
import functools
import jax
import jax.numpy as jnp
from jax.experimental import pallas as pl
from jax.experimental.pallas import tpu as pltpu

N = 10000
E = 160000
H = 512
RATIO = 0.8


def _conv_mean(x, src, dst, valid_f, Wrel, brel, Wroot):
    n = x.shape[0]
    msgs = x[src] * valid_f[:, None]
    agg = jax.ops.segment_sum(msgs, dst, num_segments=n)
    cnt = jax.ops.segment_sum(valid_f, dst, num_segments=n)
    mean = agg / jnp.maximum(cnt, 1.0)[:, None]
    return mean @ Wrel + brel + x @ Wroot


def _head_kernel(z_ref, w1_ref, b1_ref, w2_ref, b2_ref, o_ref):
    z = z_ref[...]
    h = jnp.maximum(z @ w1_ref[...] + b1_ref[...][None, :], 0.0)
    logits = h @ w2_ref[...] + b2_ref[...][None, :]
    m = jnp.max(logits, axis=-1, keepdims=True)
    e = jnp.exp(logits - m)
    lse = jnp.log(jnp.sum(e, axis=-1, keepdims=True)) + m
    o_ref[...] = logits - lse


def kernel(x, edge_index, batch, conv1_Wrel, conv1_brel, conv1_Wroot, conv2_Wrel, conv2_brel, conv2_Wroot, conv3_Wrel, conv3_brel, conv3_Wroot, pool1_w, pool2_w, lin1_W, lin1_b, lin2_W, lin2_b):
    src = edge_index[0]
    dst = edge_index[1]
    valid = jnp.ones((E,), x.dtype)

    h = jax.nn.relu(_conv_mean(x, src, dst, valid, conv1_Wrel, conv1_brel, conv1_Wroot))
    xs0 = jnp.mean(h, axis=0, keepdims=True)

    h = jax.nn.relu(_conv_mean(h, src, dst, valid, conv2_Wrel, conv2_brel, conv2_Wroot))
    xs1 = jnp.mean(h, axis=0, keepdims=True)

    score = jnp.tanh((h @ pool1_w) / jnp.linalg.norm(pool1_w))
    k = int(math.ceil(RATIO * N))
    topv, perm = jax.lax.top_k(score, k)
    keptf = jnp.zeros((N,), jnp.float32).at[perm].set(1.0)
    g = h * (score * keptf)[:, None]

    agg3 = jax.ops.segment_sum(g[src], dst, num_segments=N)
    cnt3 = jax.ops.segment_sum(keptf[src], dst, num_segments=N)
    mean3 = agg3 / jnp.maximum(cnt3, 1.0)[:, None]
    h3 = jax.nn.relu(mean3 @ conv3_Wrel + conv3_brel + g @ conv3_Wroot)
    xs2 = (jnp.sum(h3 * keptf[:, None], axis=0, keepdims=True)) / float(k)

    z = jnp.concatenate([xs0, xs1, xs2], axis=1)

    out = pl.pallas_call(
        _head_kernel,
        out_shape=jax.ShapeDtypeStruct((1, lin2_W.shape[1]), jnp.float32),
    )(z, lin1_W, lin1_b, lin2_W, lin2_b)
    return out


# speedup vs baseline: 3.1993x; 2.4229x over previous
import math
import functools
import jax
import jax.numpy as jnp
from jax import lax
from jax.experimental import pallas as pl
from jax.experimental.pallas import tpu as pltpu
from jax.experimental.pallas import tpu_sc as plsc

N = 10000
E = 160000
H = 512
RATIO = 0.8
K = int(math.ceil(RATIO * N))

NTILES = 16
EPT = E // NTILES
W = 80
NWIN = EPT // W
NPAD = 10240
RPT = NPAD // NTILES


def _make_segsum(nc: int, with_cnt: bool):
    n_out = nc + (1 if with_cnt else 0)
    out_type = [jax.ShapeDtypeStruct((NPAD, 128), jnp.float32) for _ in range(nc)]
    if with_cnt:
        out_type.append(jax.ShapeDtypeStruct((NPAD, 128), jnp.float32))

    scratch = [
        pltpu.VMEM((W,), jnp.int32),
        pltpu.VMEM((W,), jnp.int32),
        pltpu.VMEM((W, 128), jnp.float32),
        pltpu.VMEM_SHARED((NPAD, 128), jnp.float32),
        pltpu.SemaphoreType.DMA,
    ]

    mesh = plsc.VectorSubcoreMesh(core_axis_name="c", subcore_axis_name="s")

    @functools.partial(pl.kernel, out_type=tuple(out_type), mesh=mesh,
                       scratch_types=scratch)
    def seg(*refs):
        n_in = nc + (1 if with_cnt else 0) + 3
        ins = refs[:n_in]
        outs = refs[n_in:n_in + n_out]
        sidx, didx, rows, sh128, sem = refs[n_in + n_out:]
        pos = nc
        sv_hbm = ins[pos] if with_cnt else None
        pos += 1 if with_cnt else 0
        src_hbm = ins[pos]
        dst_hbm = ins[pos + 1]
        z128 = ins[pos + 2]

        cid = lax.axis_index("c")
        sid = lax.axis_index("s")
        tile_base = sid * EPT
        r0 = sid * RPT

        def do_chunk(in_hbm, out_hbm, shared, rows_v, zeros_hbm):
            pltpu.sync_copy(zeros_hbm.at[pl.ds(r0, RPT)],
                            shared.at[pl.ds(r0, RPT)])
            plsc.subcore_barrier()

            def win(w, carry):
                base = tile_base + w * W
                pltpu.sync_copy(src_hbm.at[pl.ds(base, W)], sidx)
                pltpu.sync_copy(dst_hbm.at[pl.ds(base, W)], didx)
                pltpu.async_copy(in_hbm.at[sidx], rows_v, sem).wait()
                pltpu.sync_copy(rows_v, shared.at[didx], add=True)
                return carry

            lax.fori_loop(0, NWIN, win, 0)
            plsc.subcore_barrier()
            pltpu.sync_copy(shared.at[pl.ds(r0, RPT)],
                            out_hbm.at[pl.ds(r0, RPT)])

        for c in range(nc):

            @pl.when(cid == (c % 2))
            def _(c=c):
                do_chunk(ins[c], outs[c], sh128, rows, z128)

        if with_cnt:

            @pl.when(cid == 0)
            def _():
                do_chunk(sv_hbm, outs[nc], sh128, rows, z128)

    return seg


_seg_2_cnt = _make_segsum(2, True)
_seg_4 = _make_segsum(4, False)
_seg_4_cnt = _make_segsum(4, True)


def _segsum(seg_fn, feat, src, dst, svec, nc, with_cnt):
    chunks = [feat[:, i * 128:(i + 1) * 128] for i in range(nc)]
    args = list(chunks)
    if with_cnt:
        sv128 = jnp.zeros((N, 128), jnp.float32).at[:, 0].set(svec)
        args.append(sv128)
    args += [src, dst, jnp.zeros((NPAD, 128), jnp.float32)]
    outs = seg_fn(*args)
    agg = jnp.concatenate([o[:N] for o in outs[:nc]], axis=1)
    cnt = outs[nc][:N, 0] if with_cnt else None
    return agg, cnt


def _head_kernel(z_ref, w1_ref, b1_ref, w2_ref, b2_ref, o_ref):
    z = z_ref[...]
    h = jnp.maximum(z @ w1_ref[...] + b1_ref[...][None, :], 0.0)
    logits = h @ w2_ref[...] + b2_ref[...][None, :]
    m = jnp.max(logits, axis=-1, keepdims=True)
    e = jnp.exp(logits - m)
    lse = jnp.log(jnp.sum(e, axis=-1, keepdims=True)) + m
    o_ref[...] = logits - lse


def kernel(x, edge_index, batch, conv1_Wrel, conv1_brel, conv1_Wroot, conv2_Wrel, conv2_brel, conv2_Wroot, conv3_Wrel, conv3_brel, conv3_Wroot, pool1_w, pool2_w, lin1_W, lin1_b, lin2_W, lin2_b):
    src = edge_index[0].astype(jnp.int32)
    dst = edge_index[1].astype(jnp.int32)
    ones = jnp.ones((N,), jnp.float32)

    agg1, cnt12 = _segsum(_seg_2_cnt, x, src, dst, ones, 2, True)
    inv12 = 1.0 / jnp.maximum(cnt12, 1.0)
    h = jax.nn.relu((agg1 * inv12[:, None]) @ conv1_Wrel + conv1_brel
                    + x @ conv1_Wroot)
    xs0 = jnp.mean(h, axis=0, keepdims=True)

    agg2, _ = _segsum(_seg_4, h, src, dst, None, 4, False)
    h = jax.nn.relu((agg2 * inv12[:, None]) @ conv2_Wrel + conv2_brel
                    + h @ conv2_Wroot)
    xs1 = jnp.mean(h, axis=0, keepdims=True)

    score = jnp.tanh((h @ pool1_w) / jnp.linalg.norm(pool1_w))
    topv, perm = jax.lax.top_k(score, K)
    keptf = jnp.zeros((N,), jnp.float32).at[perm].set(1.0)
    g = h * (score * keptf)[:, None]

    agg3, cnt3 = _segsum(_seg_4_cnt, g, src, dst, keptf, 4, True)
    mean3 = agg3 / jnp.maximum(cnt3, 1.0)[:, None]
    h3 = jax.nn.relu(mean3 @ conv3_Wrel + conv3_brel + g @ conv3_Wroot)
    xs2 = jnp.sum(h3 * keptf[:, None], axis=0, keepdims=True) / float(K)

    z = jnp.concatenate([xs0, xs1, xs2], axis=1)
    out = pl.pallas_call(
        _head_kernel,
        out_shape=jax.ShapeDtypeStruct((1, lin2_W.shape[1]), jnp.float32),
    )(z, lin1_W, lin1_b, lin2_W, lin2_b)
    return out


# speedup vs baseline: 7.2372x; 2.2622x over previous
import math
import functools
import jax
import jax.numpy as jnp
from jax import lax
from jax.experimental import pallas as pl
from jax.experimental.pallas import tpu as pltpu
from jax.experimental.pallas import tpu_sc as plsc

N = 10000
E = 160000
H = 512
RATIO = 0.8
K = int(math.ceil(RATIO * N))

NTILES = 16
EPT = E // NTILES
W = 80
NWIN = EPT // W
NPAD = 10240
RPT = NPAD // NTILES


def _make_segsum(nc: int, with_cnt: bool):
    n_out = nc + (1 if with_cnt else 0)
    out_type = [jax.ShapeDtypeStruct((NPAD, 128), jnp.float32) for _ in range(nc)]
    if with_cnt:
        out_type.append(jax.ShapeDtypeStruct((NPAD, 128), jnp.float32))

    scratch = [
        pltpu.VMEM((EPT,), jnp.int32),
        pltpu.VMEM((NWIN, W), jnp.int32),
        pltpu.VMEM((W, 128), jnp.float32),
        pltpu.VMEM((W, 128), jnp.float32),
        pltpu.VMEM_SHARED((NPAD, 128), jnp.float32),
        pltpu.SemaphoreType.DMA,
        pltpu.SemaphoreType.DMA,
    ]

    mesh = plsc.VectorSubcoreMesh(core_axis_name="c", subcore_axis_name="s")

    @functools.partial(pl.kernel, out_type=tuple(out_type), mesh=mesh,
                       scratch_types=scratch)
    def seg(*refs):
        n_in = nc + (1 if with_cnt else 0) + 3
        ins = refs[:n_in]
        outs = refs[n_in:n_in + n_out]
        sidx1, didx2, rowsA, rowsB, sh128, semA, semB = refs[n_in + n_out:]
        pos = nc
        sv_hbm = ins[pos] if with_cnt else None
        pos += 1 if with_cnt else 0
        src3 = ins[pos]
        dst3 = ins[pos + 1]
        z128 = ins[pos + 2]

        cid = lax.axis_index("c")
        sid = lax.axis_index("s")
        r0 = sid * RPT

        pltpu.sync_copy(src3.at[pl.ds(sid * EPT, EPT)], sidx1)
        pltpu.sync_copy(dst3.at[sid], didx2)

        def do_chunk(in_hbm, out_hbm):
            pltpu.sync_copy(z128.at[pl.ds(r0, RPT)],
                            sh128.at[pl.ds(r0, RPT)])
            plsc.subcore_barrier()

            pltpu.async_copy(in_hbm.at[sidx1.at[pl.ds(0, W)]], rowsA, semA)

            def waitA():
                pltpu.make_async_copy(in_hbm.at[sidx1.at[pl.ds(0, W)]], rowsA,
                                      semA).wait()

            def body(i, carry):
                w0 = 2 * i
                hB = pltpu.async_copy(
                    in_hbm.at[sidx1.at[pl.ds((w0 + 1) * W, W)]], rowsB, semB)
                waitA()
                pltpu.sync_copy(rowsA, sh128.at[didx2.at[w0]], add=True)
                pltpu.async_copy(
                    in_hbm.at[sidx1.at[pl.ds((w0 + 2) * W, W)]], rowsA, semA)
                hB.wait()
                pltpu.sync_copy(rowsB, sh128.at[didx2.at[w0 + 1]], add=True)
                return carry

            lax.fori_loop(0, (NWIN - 1) // 2, body, 0)
            waitA()
            pltpu.sync_copy(rowsA, sh128.at[didx2.at[NWIN - 1]], add=True)
            plsc.subcore_barrier()
            pltpu.sync_copy(sh128.at[pl.ds(r0, RPT)],
                            out_hbm.at[pl.ds(r0, RPT)])

        for c in range(nc):

            @pl.when(cid == (c % 2))
            def _(c=c):
                do_chunk(ins[c], outs[c])

        if with_cnt:

            @pl.when(cid == 0)
            def _():
                do_chunk(sv_hbm, outs[nc])

    return seg


_seg_2_cnt = _make_segsum(2, True)
_seg_4 = _make_segsum(4, False)
_seg_4_cnt = _make_segsum(4, True)


def _segsum(seg_fn, feat, src, dst, svec, nc, with_cnt):
    chunks = [feat[:, i * 128:(i + 1) * 128] for i in range(nc)]
    args = list(chunks)
    if with_cnt:
        sv128 = jnp.zeros((N, 128), jnp.float32).at[:, 0].set(svec)
        args.append(sv128)
    args += [src, dst.reshape(NTILES, NWIN, W),
             jnp.zeros((NPAD, 128), jnp.float32)]
    outs = seg_fn(*args)
    agg = jnp.concatenate([o[:N] for o in outs[:nc]], axis=1)
    cnt = outs[nc][:N, 0] if with_cnt else None
    return agg, cnt


def _head_kernel(z_ref, w1_ref, b1_ref, w2_ref, b2_ref, o_ref):
    z = z_ref[...]
    h = jnp.maximum(z @ w1_ref[...] + b1_ref[...][None, :], 0.0)
    logits = h @ w2_ref[...] + b2_ref[...][None, :]
    m = jnp.max(logits, axis=-1, keepdims=True)
    e = jnp.exp(logits - m)
    lse = jnp.log(jnp.sum(e, axis=-1, keepdims=True)) + m
    o_ref[...] = logits - lse


def kernel(x, edge_index, batch, conv1_Wrel, conv1_brel, conv1_Wroot, conv2_Wrel, conv2_brel, conv2_Wroot, conv3_Wrel, conv3_brel, conv3_Wroot, pool1_w, pool2_w, lin1_W, lin1_b, lin2_W, lin2_b):
    src = edge_index[0].astype(jnp.int32)
    dst = edge_index[1].astype(jnp.int32)
    ones = jnp.ones((N,), jnp.float32)

    agg1, cnt12 = _segsum(_seg_2_cnt, x, src, dst, ones, 2, True)
    inv12 = 1.0 / jnp.maximum(cnt12, 1.0)
    h = jax.nn.relu((agg1 * inv12[:, None]) @ conv1_Wrel + conv1_brel
                    + x @ conv1_Wroot)
    xs0 = jnp.mean(h, axis=0, keepdims=True)

    agg2, _ = _segsum(_seg_4, h, src, dst, None, 4, False)
    h = jax.nn.relu((agg2 * inv12[:, None]) @ conv2_Wrel + conv2_brel
                    + h @ conv2_Wroot)
    xs1 = jnp.mean(h, axis=0, keepdims=True)

    score = jnp.tanh((h @ pool1_w) / jnp.linalg.norm(pool1_w))
    topv, perm = jax.lax.top_k(score, K)
    keptf = jnp.zeros((N,), jnp.float32).at[perm].set(1.0)
    g = h * (score * keptf)[:, None]

    agg3, cnt3 = _segsum(_seg_4_cnt, g, src, dst, keptf, 4, True)
    mean3 = agg3 / jnp.maximum(cnt3, 1.0)[:, None]
    h3 = jax.nn.relu(mean3 @ conv3_Wrel + conv3_brel + g @ conv3_Wroot)
    xs2 = jnp.sum(h3 * keptf[:, None], axis=0, keepdims=True) / float(K)

    z = jnp.concatenate([xs0, xs1, xs2], axis=1)
    out = pl.pallas_call(
        _head_kernel,
        out_shape=jax.ShapeDtypeStruct((1, lin2_W.shape[1]), jnp.float32),
    )(z, lin1_W, lin1_b, lin2_W, lin2_b)
    return out


# speedup vs baseline: 7.7952x; 1.0771x over previous
import math
import functools
import jax
import jax.numpy as jnp
from jax import lax
from jax.experimental import pallas as pl
from jax.experimental.pallas import tpu as pltpu
from jax.experimental.pallas import tpu_sc as plsc

N = 10000
E = 160000
H = 512
RATIO = 0.8
K = int(math.ceil(RATIO * N))

NTILES = 16
EPT = E // NTILES
W = 80
NWIN = EPT // W
NPAD = 10240
RPT = NPAD // NTILES


def _make_segsum(nc: int, with_cnt: bool):
    n_out = nc + (1 if with_cnt else 0)
    out_type = [jax.ShapeDtypeStruct((NPAD, 128), jnp.float32) for _ in range(nc)]
    if with_cnt:
        out_type.append(jax.ShapeDtypeStruct((NPAD, 128), jnp.float32))

    scratch = [
        pltpu.VMEM((EPT,), jnp.int32),
        pltpu.VMEM((NWIN, W), jnp.int32),
        pltpu.VMEM((W, 128), jnp.float32),
        pltpu.VMEM((W, 128), jnp.float32),
        pltpu.VMEM_SHARED((NPAD, 128), jnp.float32),
        pltpu.SemaphoreType.DMA,
        pltpu.SemaphoreType.DMA,
    ]

    mesh = plsc.VectorSubcoreMesh(core_axis_name="c", subcore_axis_name="s")

    @functools.partial(pl.kernel, out_type=tuple(out_type), mesh=mesh,
                       scratch_types=scratch)
    def seg(*refs):
        n_in = nc + (1 if with_cnt else 0) + 3
        ins = refs[:n_in]
        outs = refs[n_in:n_in + n_out]
        sidx1, didx2, rowsA, rowsB, sh128, semA, semB = refs[n_in + n_out:]
        pos = nc
        sv_hbm = ins[pos] if with_cnt else None
        pos += 1 if with_cnt else 0
        src3 = ins[pos]
        dst3 = ins[pos + 1]
        z128 = ins[pos + 2]

        cid = lax.axis_index("c")
        sid = lax.axis_index("s")
        r0 = sid * RPT

        pltpu.sync_copy(src3.at[pl.ds(sid * EPT, EPT)], sidx1)
        pltpu.sync_copy(dst3.at[sid], didx2)

        def do_chunk(in_hbm, out_hbm):
            pltpu.sync_copy(z128.at[pl.ds(r0, RPT)],
                            sh128.at[pl.ds(r0, RPT)])
            plsc.subcore_barrier()

            pltpu.async_copy(in_hbm.at[sidx1.at[pl.ds(0, W)]], rowsA, semA)

            def waitA():
                pltpu.make_async_copy(in_hbm.at[sidx1.at[pl.ds(0, W)]], rowsA,
                                      semA).wait()

            def body(i, carry):
                w0 = 2 * i
                hB = pltpu.async_copy(
                    in_hbm.at[sidx1.at[pl.ds((w0 + 1) * W, W)]], rowsB, semB)
                waitA()
                pltpu.sync_copy(rowsA, sh128.at[didx2.at[w0]], add=True)
                pltpu.async_copy(
                    in_hbm.at[sidx1.at[pl.ds((w0 + 2) * W, W)]], rowsA, semA)
                hB.wait()
                pltpu.sync_copy(rowsB, sh128.at[didx2.at[w0 + 1]], add=True)
                return carry

            lax.fori_loop(0, (NWIN - 1) // 2, body, 0)
            waitA()
            pltpu.sync_copy(rowsA, sh128.at[didx2.at[NWIN - 1]], add=True)
            plsc.subcore_barrier()
            pltpu.sync_copy(sh128.at[pl.ds(r0, RPT)],
                            out_hbm.at[pl.ds(r0, RPT)])

        for c in range(nc):

            @pl.when(cid == (c % 2))
            def _(c=c):
                do_chunk(ins[c], outs[c])

        if with_cnt:

            @pl.when(cid == 0)
            def _():
                do_chunk(sv_hbm, outs[nc])

    return seg


_seg_2_cnt = _make_segsum(2, True)
_seg_4 = _make_segsum(4, False)
_seg_4_cnt = _make_segsum(4, True)


def _segsum(seg_fn, chunks, src, dst, sv128):
    args = list(chunks)
    if sv128 is not None:
        args.append(sv128)
    args += [src, dst.reshape(NTILES, NWIN, W),
             jnp.zeros((NPAD, 128), jnp.float32)]
    return seg_fn(*args)


BLK = 1024
GRID = NPAD // BLK
NV = NPAD // 128


def _conv_body(nc, use_score, mask_kind, out_h):
    def body(*refs):
        i = pl.program_id(0)
        pos = 0
        aggs = refs[pos:pos + nc]; pos += nc
        cnt = refs[pos]; pos += 1
        xins = refs[pos:pos + nc]; pos += nc
        wrel = refs[pos]; brel = refs[pos + 1]; wroot = refs[pos + 2]
        pos += 3
        wsc = refs[pos]; pos += 1
        kept = None
        if mask_kind == "kept":
            kept = refs[pos]; pos += 1
        outs = list(refs[pos:])
        o = 0
        h_out = None
        if out_h:
            h_out = outs[o:o + 4]; o += 4
        cs_ref = outs[o]; o += 1
        sdot_ref = outs[o] if use_score else None

        inv = 1.0 / jnp.maximum(cnt[...][:, 0:1], 1.0)
        acc = jnp.zeros((BLK, H), jnp.float32)
        for c in range(nc):
            acc += (aggs[c][...] * inv) @ wrel[pl.ds(c * 128, 128), :]
            acc += xins[c][...] @ wroot[pl.ds(c * 128, 128), :]
        hv = jnp.maximum(acc + brel[...], 0.0)
        if out_h:
            for c in range(4):
                h_out[c][...] = hv[:, c * 128:(c + 1) * 128]
        if mask_kind == "kept":
            m = kept[...]
        else:
            rows = jax.lax.broadcasted_iota(jnp.int32, (BLK, 1), 0) + i * BLK
            m = (rows < N).astype(jnp.float32)
        cs = jnp.sum(hv * m, axis=0, keepdims=True)

        @pl.when(i == 0)
        def _():
            cs_ref[...] = jnp.zeros_like(cs_ref)

        cs_ref[...] += cs
        if use_score:
            sdot_ref[...] = jnp.sum(hv * wsc[...], axis=1, keepdims=True)

    return body


def _conv_tc(aggs, cnt, xins, Wrel, brel, Wroot, wsc, kept, use_score,
             mask_kind, out_h):
    nc = len(aggs)
    fin = nc * 128
    chunk_spec = pl.BlockSpec((BLK, 128), lambda i: (i, 0))
    col_spec = pl.BlockSpec((BLK, 1), lambda i: (i, 0))
    full = lambda a: pl.BlockSpec(a.shape, lambda i: (0, 0))
    in_specs = ([chunk_spec] * nc + [chunk_spec] + [chunk_spec] * nc
                + [full(Wrel), full(brel), full(Wroot), full(wsc)])
    args = list(aggs) + [cnt] + list(xins) + [Wrel, brel, Wroot, wsc]
    if mask_kind == "kept":
        in_specs.append(col_spec)
        args.append(kept)
    out_shape = []
    out_specs = []
    if out_h:
        out_shape += [jax.ShapeDtypeStruct((NPAD, 128), jnp.float32)] * 4
        out_specs += [chunk_spec] * 4
    out_shape.append(jax.ShapeDtypeStruct((1, H), jnp.float32))
    out_specs.append(pl.BlockSpec((1, H), lambda i: (0, 0)))
    if use_score:
        out_shape.append(jax.ShapeDtypeStruct((NPAD, 1), jnp.float32))
        out_specs.append(col_spec)
    return pl.pallas_call(
        _conv_body(nc, use_score, mask_kind, out_h),
        grid=(GRID,),
        in_specs=in_specs,
        out_specs=out_specs,
        out_shape=out_shape,
    )(*args)


def _topk_body(sdot_ref, wsc_ref, scale_ref, kept_ref):
    w = wsc_ref[...]
    rin = jax.lax.rsqrt(jnp.sum(w * w))
    score = jnp.tanh(sdot_ref[...] * rin)
    valid = jax.lax.broadcasted_iota(jnp.int32, (NPAD, 1), 0) < N
    bits = jax.lax.bitcast_convert_type(score, jnp.int32)
    minint = jnp.int32(-2147483648)
    u = jnp.where(bits < 0, ~bits, bits | minint)
    svals = jnp.where(valid, u ^ minint, minint)

    def bsearch(j, tu):
        cand = tu | jax.lax.shift_left(jnp.int32(1), 31 - j)
        cnt = jnp.sum(jnp.where(svals >= (cand ^ minint), 1, 0))
        return jnp.where(cnt >= K, cand, tu)

    tu = jax.lax.fori_loop(0, 32, bsearch, jnp.int32(0))
    ts = tu ^ minint
    n_gt = jnp.sum(jnp.where(svals > ts, 1, 0))
    m = K - n_gt
    ties = (svals == ts).astype(jnp.int32)
    rank = ties
    sh = 1
    while sh < NPAD:
        rank = rank + jnp.concatenate(
            [jnp.zeros((sh, 1), jnp.int32), rank[:-sh]], axis=0)
        sh *= 2
    rank = rank - ties
    keptb = (svals > ts) | ((ties > 0) & (rank < m))
    keptf = keptb.astype(jnp.float32)
    kept_ref[...] = keptf
    scale_ref[...] = score * keptf


def _topk_tc(sdot, wsc):
    return pl.pallas_call(
        _topk_body,
        out_shape=[jax.ShapeDtypeStruct((NPAD, 1), jnp.float32)] * 2,
    )(sdot, wsc)


def _scaleg_body(h0, h1, h2, h3, scale, kept, g0, g1, g2, g3, sv):
    sc = scale[...]
    for hin, gout in ((h0, g0), (h1, g1), (h2, g2), (h3, g3)):
        gout[...] = hin[...] * sc
    lane = jax.lax.broadcasted_iota(jnp.int32, (BLK, 128), 1)
    sv[...] = jnp.where(lane == 0, kept[...], 0.0)


def _scaleg_tc(hcs, scale, kept):
    chunk_spec = pl.BlockSpec((BLK, 128), lambda i: (i, 0))
    col_spec = pl.BlockSpec((BLK, 1), lambda i: (i, 0))
    return pl.pallas_call(
        _scaleg_body,
        grid=(GRID,),
        in_specs=[chunk_spec] * 4 + [col_spec, col_spec],
        out_specs=[chunk_spec] * 5,
        out_shape=[jax.ShapeDtypeStruct((NPAD, 128), jnp.float32)] * 5,
    )(*hcs, scale, kept)


def _head_body(cs1, cs2, cs3, w1, b1, w2, b2, o_ref):
    z1 = cs1[...] * (1.0 / N)
    z2 = cs2[...] * (1.0 / N)
    z3 = cs3[...] * (1.0 / K)
    zh = (z1 @ w1[pl.ds(0, H), :] + z2 @ w1[pl.ds(H, H), :]
          + z3 @ w1[pl.ds(2 * H, H), :])
    hh = jnp.maximum(zh + b1[...], 0.0)
    logits = hh @ w2[...] + b2[...]
    mx = jnp.max(logits, axis=-1, keepdims=True)
    lse = jnp.log(jnp.sum(jnp.exp(logits - mx), axis=-1, keepdims=True)) + mx
    o_ref[...] = logits - lse


def _head_tc(cs1, cs2, cs3, w1, b1, w2, b2):
    return pl.pallas_call(
        _head_body,
        out_shape=jax.ShapeDtypeStruct((1, w2.shape[1]), jnp.float32),
    )(cs1, cs2, cs3, w1, b1, w2, b2)


def kernel(x, edge_index, batch, conv1_Wrel, conv1_brel, conv1_Wroot, conv2_Wrel, conv2_brel, conv2_Wroot, conv3_Wrel, conv3_brel, conv3_Wroot, pool1_w, pool2_w, lin1_W, lin1_b, lin2_W, lin2_b):
    src = edge_index[0].astype(jnp.int32)
    dst = edge_index[1].astype(jnp.int32)

    xp = jnp.pad(x, ((0, NPAD - N), (0, 0)))
    xc = [xp[:, 0:128], xp[:, 128:256]]
    ones128 = jnp.zeros((NPAD, 128), jnp.float32).at[:N, 0].set(1.0)
    b1 = conv1_brel.reshape(1, H)
    b2 = conv2_brel.reshape(1, H)
    b3 = conv3_brel.reshape(1, H)
    wsc = pool1_w.reshape(1, H)
    lb1 = lin1_b.reshape(1, H)
    lb2 = lin2_b.reshape(1, lin2_W.shape[1])

    o = _segsum(_seg_2_cnt, xc, src, dst, ones128)
    agg1 = o[:2]
    cnt12 = o[2]
    h1_0, h1_1, h1_2, h1_3, cs1 = _conv_tc(
        agg1, cnt12, xc, conv1_Wrel, b1, conv1_Wroot, wsc, None,
        use_score=False, mask_kind="iota", out_h=True)
    h1 = [h1_0, h1_1, h1_2, h1_3]

    agg2 = _segsum(_seg_4, h1, src, dst, None)
    h2_0, h2_1, h2_2, h2_3, cs2, sdot = _conv_tc(
        agg2, cnt12, h1, conv2_Wrel, b2, conv2_Wroot, wsc, None,
        use_score=True, mask_kind="iota", out_h=True)
    h2 = [h2_0, h2_1, h2_2, h2_3]

    scale, kept = _topk_tc(sdot, wsc)
    g0, g1, g2, g3, sv128 = _scaleg_tc(h2, scale, kept)

    o = _segsum(_seg_4_cnt, [g0, g1, g2, g3], src, dst, sv128)
    agg3 = o[:4]
    cnt3 = o[4]
    (cs3,) = _conv_tc(
        agg3, cnt3, [g0, g1, g2, g3], conv3_Wrel, b3, conv3_Wroot, wsc, kept,
        use_score=False, mask_kind="kept", out_h=False)

    return _head_tc(cs1, cs2, cs3, lin1_W, lb1, lin2_W, lb2)


# speedup vs baseline: 8.8186x; 1.1313x over previous
import math
import functools
import jax
import jax.numpy as jnp
from jax import lax
from jax.experimental import pallas as pl
from jax.experimental.pallas import tpu as pltpu
from jax.experimental.pallas import tpu_sc as plsc

N = 10000
E = 160000
H = 512
RATIO = 0.8
K = int(math.ceil(RATIO * N))

NTILES = 16
EPT = E // NTILES
W = 80
NWIN = EPT // W
NPAD = 10240
RPT = NPAD // NTILES


def _make_segsum(nc: int, cnt_mode):
    with_sv = cnt_mode == "gather"
    with_cnt = cnt_mode is not None
    n_out = nc + (2 if with_cnt else 0)
    out_type = [jax.ShapeDtypeStruct((NPAD, 128), jnp.float32)
                for _ in range(n_out)]

    scratch = [
        pltpu.VMEM((EPT,), jnp.int32),
        pltpu.VMEM((NWIN, W), jnp.int32),
        pltpu.VMEM((W, 128), jnp.float32),
        pltpu.VMEM((W, 128), jnp.float32),
        pltpu.VMEM_SHARED((NPAD, 128), jnp.float32),
        pltpu.SemaphoreType.DMA,
        pltpu.SemaphoreType.DMA,
    ]

    mesh = plsc.VectorSubcoreMesh(core_axis_name="c", subcore_axis_name="s")

    @functools.partial(pl.kernel, out_type=tuple(out_type), mesh=mesh,
                       scratch_types=scratch)
    def seg(*refs):
        n_in = nc + (1 if with_sv else 0) + 4
        ins = refs[:n_in]
        outs = refs[n_in:n_in + n_out]
        sidx1, didx2, rowsA, rowsB, sh128, semA, semB = refs[n_in + n_out:]
        pos = nc
        sv_hbm = ins[pos] if with_sv else None
        pos += 1 if with_sv else 0
        src1 = ins[pos]
        dst3 = ins[pos + 1]
        z128 = ins[pos + 2]
        ones_hbm = ins[pos + 3]

        cid = lax.axis_index("c")
        sid = lax.axis_index("s")
        r0 = sid * RPT

        pltpu.sync_copy(src1.at[pl.ds(sid * EPT, EPT)], sidx1)
        pltpu.sync_copy(dst3.at[sid], didx2)

        def waitA(in_hbm):
            pltpu.make_async_copy(in_hbm.at[sidx1.at[pl.ds(0, W)]], rowsA,
                                  semA).wait()

        def waitB(in_hbm):
            pltpu.make_async_copy(in_hbm.at[sidx1.at[pl.ds(0, W)]], rowsB,
                                  semB).wait()

        def accum(in_hbm, w_lo, nw):
            pltpu.async_copy(in_hbm.at[sidx1.at[pl.ds(w_lo * W, W)]], rowsA,
                             semA)

            def body(i, carry):
                w0 = w_lo + 2 * i

                @pl.when(2 * i + 1 < nw)
                def _():
                    pltpu.async_copy(
                        in_hbm.at[sidx1.at[pl.ds((w0 + 1) * W, W)]], rowsB,
                        semB)

                waitA(in_hbm)
                pltpu.sync_copy(rowsA, sh128.at[didx2.at[w0]], add=True)

                @pl.when(2 * i + 2 < nw)
                def _():
                    pltpu.async_copy(
                        in_hbm.at[sidx1.at[pl.ds((w0 + 2) * W, W)]], rowsA,
                        semA)

                @pl.when(2 * i + 1 < nw)
                def _():
                    waitB(in_hbm)
                    pltpu.sync_copy(rowsB, sh128.at[didx2.at[w0 + 1]],
                                    add=True)

                return carry

            lax.fori_loop(0, (nw + 1) // 2, body, 0)

        def accum_ones(w_lo, nw):
            pltpu.sync_copy(ones_hbm, rowsA)

            def body(i, carry):
                pltpu.sync_copy(rowsA, sh128.at[didx2.at[w_lo + i]],
                                add=True)
                return carry

            lax.fori_loop(0, nw, body, 0)

        def zero_own():
            pltpu.sync_copy(z128.at[pl.ds(r0, RPT)], sh128.at[pl.ds(r0, RPT)])
            plsc.subcore_barrier()

        def flush_own(out_hbm):
            plsc.subcore_barrier()
            pltpu.sync_copy(sh128.at[pl.ds(r0, RPT)],
                            out_hbm.at[pl.ds(r0, RPT)])

        def do_chunk(in_hbm, out_hbm):
            zero_own()
            accum(in_hbm, 0, NWIN)
            flush_own(out_hbm)

        for c in range(nc):

            @pl.when(cid == (c % 2))
            def _(c=c):
                do_chunk(ins[c], outs[c])

        if with_cnt:
            half = (NWIN + 1) // 2
            for core, w_lo, nw in ((0, 0, half), (1, half, NWIN - half)):

                @pl.when(cid == core)
                def _(w_lo=w_lo, nw=nw, out=outs[nc + core]):
                    zero_own()
                    if with_sv:
                        accum(sv_hbm, w_lo, nw)
                    else:
                        accum_ones(w_lo, nw)
                    flush_own(out)

    return seg


_seg_2_cnt = _make_segsum(2, "ones")
_seg_4 = _make_segsum(4, None)
_seg_4_cnt = _make_segsum(4, "gather")


def _segsum(seg_fn, chunks, src, dst, sv128):
    args = list(chunks)
    if sv128 is not None:
        args.append(sv128)
    args += [src, dst.reshape(NTILES, NWIN, W),
             jnp.zeros((NPAD, 128), jnp.float32),
             jnp.ones((W, 128), jnp.float32)]
    return seg_fn(*args)


BLK = 1024
GRID = NPAD // BLK
NV = NPAD // 128


def _conv_body(nc, use_score, mask_kind, out_h):
    def body(*refs):
        i = pl.program_id(0)
        pos = 0
        aggs = refs[pos:pos + nc]; pos += nc
        cntA = refs[pos]; cntB = refs[pos + 1]; pos += 2
        xins = refs[pos:pos + nc]; pos += nc
        wrel = refs[pos]; brel = refs[pos + 1]; wroot = refs[pos + 2]
        pos += 3
        wsc = refs[pos]; pos += 1
        kept = None
        if mask_kind == "kept":
            kept = refs[pos]; pos += 1
        outs = list(refs[pos:])
        o = 0
        h_out = None
        if out_h:
            h_out = outs[o:o + 4]; o += 4
        cs_ref = outs[o]; o += 1
        sdot_ref = outs[o] if use_score else None

        inv = 1.0 / jnp.maximum(cntA[...][:, 0:1] + cntB[...][:, 0:1], 1.0)
        acc = jnp.zeros((BLK, H), jnp.float32)
        for c in range(nc):
            acc += (aggs[c][...] * inv) @ wrel[pl.ds(c * 128, 128), :]
            acc += xins[c][...] @ wroot[pl.ds(c * 128, 128), :]
        hv = jnp.maximum(acc + brel[...], 0.0)
        if out_h:
            for c in range(4):
                h_out[c][...] = hv[:, c * 128:(c + 1) * 128]
        if mask_kind == "kept":
            m = kept[...]
        else:
            rows = jax.lax.broadcasted_iota(jnp.int32, (BLK, 1), 0) + i * BLK
            m = (rows < N).astype(jnp.float32)
        cs = jnp.sum(hv * m, axis=0, keepdims=True)

        @pl.when(i == 0)
        def _():
            cs_ref[...] = jnp.zeros_like(cs_ref)

        cs_ref[...] += cs
        if use_score:
            sdot_ref[...] = jnp.sum(hv * wsc[...], axis=1, keepdims=True)

    return body


def _conv_tc(aggs, cnts, xins, Wrel, brel, Wroot, wsc, kept, use_score,
             mask_kind, out_h):
    nc = len(aggs)
    chunk_spec = pl.BlockSpec((BLK, 128), lambda i: (i, 0))
    col_spec = pl.BlockSpec((BLK, 1), lambda i: (i, 0))
    full = lambda a: pl.BlockSpec(a.shape, lambda i: (0, 0))
    in_specs = ([chunk_spec] * nc + [chunk_spec, chunk_spec]
                + [chunk_spec] * nc
                + [full(Wrel), full(brel), full(Wroot), full(wsc)])
    args = list(aggs) + list(cnts) + list(xins) + [Wrel, brel, Wroot, wsc]
    if mask_kind == "kept":
        in_specs.append(col_spec)
        args.append(kept)
    out_shape = []
    out_specs = []
    if out_h:
        out_shape += [jax.ShapeDtypeStruct((NPAD, 128), jnp.float32)] * 4
        out_specs += [chunk_spec] * 4
    out_shape.append(jax.ShapeDtypeStruct((1, H), jnp.float32))
    out_specs.append(pl.BlockSpec((1, H), lambda i: (0, 0)))
    if use_score:
        out_shape.append(jax.ShapeDtypeStruct((NPAD, 1), jnp.float32))
        out_specs.append(col_spec)
    return pl.pallas_call(
        _conv_body(nc, use_score, mask_kind, out_h),
        grid=(GRID,),
        in_specs=in_specs,
        out_specs=out_specs,
        out_shape=out_shape,
    )(*args)


def _topk_body(sdot_ref, wsc_ref, scale_ref, kept_ref):
    w = wsc_ref[...]
    rin = jax.lax.rsqrt(jnp.sum(w * w))
    score = jnp.tanh(sdot_ref[...] * rin)
    valid = jax.lax.broadcasted_iota(jnp.int32, (NPAD, 1), 0) < N
    bits = jax.lax.bitcast_convert_type(score, jnp.int32)
    minint = jnp.int32(-2147483648)
    u = jnp.where(bits < 0, ~bits, bits | minint)
    svals = jnp.where(valid, u ^ minint, minint)

    def bsearch(j, tu):
        cand = tu | jax.lax.shift_left(jnp.int32(1), 31 - j)
        cnt = jnp.sum(jnp.where(svals >= (cand ^ minint), 1, 0))
        return jnp.where(cnt >= K, cand, tu)

    tu = jax.lax.fori_loop(0, 32, bsearch, jnp.int32(0))
    ts = tu ^ minint
    n_gt = jnp.sum(jnp.where(svals > ts, 1, 0))
    m = K - n_gt
    ties = (svals == ts).astype(jnp.int32)
    rank = ties
    sh = 1
    while sh < NPAD:
        rank = rank + jnp.concatenate(
            [jnp.zeros((sh, 1), jnp.int32), rank[:-sh]], axis=0)
        sh *= 2
    rank = rank - ties
    keptb = (svals > ts) | ((ties > 0) & (rank < m))
    keptf = keptb.astype(jnp.float32)
    kept_ref[...] = keptf
    scale_ref[...] = score * keptf


def _topk_tc(sdot, wsc):
    return pl.pallas_call(
        _topk_body,
        out_shape=[jax.ShapeDtypeStruct((NPAD, 1), jnp.float32)] * 2,
    )(sdot, wsc)


def _scaleg_body(h0, h1, h2, h3, scale, kept, g0, g1, g2, g3, sv):
    sc = scale[...]
    for hin, gout in ((h0, g0), (h1, g1), (h2, g2), (h3, g3)):
        gout[...] = hin[...] * sc
    lane = jax.lax.broadcasted_iota(jnp.int32, (BLK, 128), 1)
    sv[...] = jnp.where(lane == 0, kept[...], 0.0)


def _scaleg_tc(hcs, scale, kept):
    chunk_spec = pl.BlockSpec((BLK, 128), lambda i: (i, 0))
    col_spec = pl.BlockSpec((BLK, 1), lambda i: (i, 0))
    return pl.pallas_call(
        _scaleg_body,
        grid=(GRID,),
        in_specs=[chunk_spec] * 4 + [col_spec, col_spec],
        out_specs=[chunk_spec] * 5,
        out_shape=[jax.ShapeDtypeStruct((NPAD, 128), jnp.float32)] * 5,
    )(*hcs, scale, kept)


def _head_body(cs1, cs2, cs3, w1, b1, w2, b2, o_ref):
    z1 = cs1[...] * (1.0 / N)
    z2 = cs2[...] * (1.0 / N)
    z3 = cs3[...] * (1.0 / K)
    zh = (z1 @ w1[pl.ds(0, H), :] + z2 @ w1[pl.ds(H, H), :]
          + z3 @ w1[pl.ds(2 * H, H), :])
    hh = jnp.maximum(zh + b1[...], 0.0)
    logits = hh @ w2[...] + b2[...]
    mx = jnp.max(logits, axis=-1, keepdims=True)
    lse = jnp.log(jnp.sum(jnp.exp(logits - mx), axis=-1, keepdims=True)) + mx
    o_ref[...] = logits - lse


def _head_tc(cs1, cs2, cs3, w1, b1, w2, b2):
    return pl.pallas_call(
        _head_body,
        out_shape=jax.ShapeDtypeStruct((1, w2.shape[1]), jnp.float32),
    )(cs1, cs2, cs3, w1, b1, w2, b2)


def kernel(x, edge_index, batch, conv1_Wrel, conv1_brel, conv1_Wroot, conv2_Wrel, conv2_brel, conv2_Wroot, conv3_Wrel, conv3_brel, conv3_Wroot, pool1_w, pool2_w, lin1_W, lin1_b, lin2_W, lin2_b):
    src = edge_index[0].astype(jnp.int32)
    dst = edge_index[1].astype(jnp.int32)

    xp = jnp.pad(x, ((0, NPAD - N), (0, 0)))
    xc = [xp[:, 0:128], xp[:, 128:256]]
    b1 = conv1_brel.reshape(1, H)
    b2 = conv2_brel.reshape(1, H)
    b3 = conv3_brel.reshape(1, H)
    wsc = pool1_w.reshape(1, H)
    lb1 = lin1_b.reshape(1, H)
    lb2 = lin2_b.reshape(1, lin2_W.shape[1])

    o = _segsum(_seg_2_cnt, xc, src, dst, None)
    agg1 = o[:2]
    cnt12 = o[2:4]
    h1_0, h1_1, h1_2, h1_3, cs1 = _conv_tc(
        agg1, cnt12, xc, conv1_Wrel, b1, conv1_Wroot, wsc, None,
        use_score=False, mask_kind="iota", out_h=True)
    h1 = [h1_0, h1_1, h1_2, h1_3]

    agg2 = _segsum(_seg_4, h1, src, dst, None)
    h2_0, h2_1, h2_2, h2_3, cs2, sdot = _conv_tc(
        agg2, cnt12, h1, conv2_Wrel, b2, conv2_Wroot, wsc, None,
        use_score=True, mask_kind="iota", out_h=True)
    h2 = [h2_0, h2_1, h2_2, h2_3]

    scale, kept = _topk_tc(sdot, wsc)
    g0, g1, g2, g3, sv128 = _scaleg_tc(h2, scale, kept)

    o = _segsum(_seg_4_cnt, [g0, g1, g2, g3], src, dst, sv128)
    agg3 = o[:4]
    cnt3 = o[4:6]
    (cs3,) = _conv_tc(
        agg3, cnt3, [g0, g1, g2, g3], conv3_Wrel, b3, conv3_Wroot, wsc, kept,
        use_score=False, mask_kind="kept", out_h=False)

    return _head_tc(cs1, cs2, cs3, lin1_W, lb1, lin2_W, lb2)


# speedup vs baseline: 9.2444x; 1.0483x over previous
import math
import functools
import jax
import jax.numpy as jnp
from jax import lax
from jax.experimental import pallas as pl
from jax.experimental.pallas import tpu as pltpu
from jax.experimental.pallas import tpu_sc as plsc

N = 10000
E = 160000
H = 512
RATIO = 0.8
K = int(math.ceil(RATIO * N))

NTILES = 16
EPT = E // NTILES
W = 80
NWIN = EPT // W
NPAD = 10240
RPT = NPAD // NTILES


def _make_segsum(nc: int, cnt_mode):
    with_sv = cnt_mode == "gather"
    with_cnt = cnt_mode is not None
    n_out = nc + (2 if with_cnt else 0)
    out_type = [jax.ShapeDtypeStruct((NPAD, 128), jnp.float32)
                for _ in range(n_out)]

    scratch = [
        pltpu.VMEM((EPT,), jnp.int32),
        pltpu.VMEM((NWIN, W), jnp.int32),
        pltpu.VMEM((W, 128), jnp.float32),
        pltpu.VMEM((W, 128), jnp.float32),
        pltpu.VMEM_SHARED((NPAD, 128), jnp.float32),
        pltpu.SemaphoreType.DMA,
        pltpu.SemaphoreType.DMA,
    ]

    mesh = plsc.VectorSubcoreMesh(core_axis_name="c", subcore_axis_name="s")

    @functools.partial(pl.kernel, out_type=tuple(out_type), mesh=mesh,
                       scratch_types=scratch)
    def seg(*refs):
        n_in = nc + (1 if with_sv else 0) + 4
        ins = refs[:n_in]
        outs = refs[n_in:n_in + n_out]
        sidx1, didx2, rowsA, rowsB, sh128, semA, semB = refs[n_in + n_out:]
        pos = nc
        sv_hbm = ins[pos] if with_sv else None
        pos += 1 if with_sv else 0
        src1 = ins[pos]
        dst3 = ins[pos + 1]
        z128 = ins[pos + 2]
        ones_hbm = ins[pos + 3]

        cid = lax.axis_index("c")
        sid = lax.axis_index("s")
        r0 = sid * RPT

        pltpu.sync_copy(src1.at[pl.ds(sid * EPT, EPT)], sidx1)
        pltpu.sync_copy(dst3.at[sid], didx2)

        def waitA(in_hbm):
            pltpu.make_async_copy(in_hbm.at[sidx1.at[pl.ds(0, W)]], rowsA,
                                  semA).wait()

        def waitB(in_hbm):
            pltpu.make_async_copy(in_hbm.at[sidx1.at[pl.ds(0, W)]], rowsB,
                                  semB).wait()

        def accum(in_hbm, w_lo, nw):
            pltpu.async_copy(in_hbm.at[sidx1.at[pl.ds(w_lo * W, W)]], rowsA,
                             semA)

            def body(i, carry):
                w0 = w_lo + 2 * i

                @pl.when(2 * i + 1 < nw)
                def _():
                    pltpu.async_copy(
                        in_hbm.at[sidx1.at[pl.ds((w0 + 1) * W, W)]], rowsB,
                        semB)

                waitA(in_hbm)
                pltpu.sync_copy(rowsA, sh128.at[didx2.at[w0]], add=True)

                @pl.when(2 * i + 2 < nw)
                def _():
                    pltpu.async_copy(
                        in_hbm.at[sidx1.at[pl.ds((w0 + 2) * W, W)]], rowsA,
                        semA)

                @pl.when(2 * i + 1 < nw)
                def _():
                    waitB(in_hbm)
                    pltpu.sync_copy(rowsB, sh128.at[didx2.at[w0 + 1]],
                                    add=True)

                return carry

            lax.fori_loop(0, (nw + 1) // 2, body, 0)

        def accum_ones(w_lo, nw):
            pltpu.sync_copy(ones_hbm, rowsA)

            def body(i, carry):
                pltpu.sync_copy(rowsA, sh128.at[didx2.at[w_lo + i]],
                                add=True)
                return carry

            lax.fori_loop(0, nw, body, 0)

        def zero_own():
            pltpu.sync_copy(z128.at[pl.ds(r0, RPT)], sh128.at[pl.ds(r0, RPT)])
            plsc.subcore_barrier()

        def flush_own(out_hbm):
            plsc.subcore_barrier()
            pltpu.sync_copy(sh128.at[pl.ds(r0, RPT)],
                            out_hbm.at[pl.ds(r0, RPT)])

        def do_chunk(in_hbm, out_hbm):
            zero_own()
            accum(in_hbm, 0, NWIN)
            flush_own(out_hbm)

        for c in range(nc):

            @pl.when(cid == (c % 2))
            def _(c=c):
                do_chunk(ins[c], outs[c])

        if with_cnt:
            half = (NWIN + 1) // 2
            for core, w_lo, nw in ((0, 0, half), (1, half, NWIN - half)):

                @pl.when(cid == core)
                def _(w_lo=w_lo, nw=nw, out=outs[nc + core]):
                    zero_own()
                    if with_sv:
                        accum(sv_hbm, w_lo, nw)
                    else:
                        accum_ones(w_lo, nw)
                    flush_own(out)

    return seg


_seg_2_cnt = _make_segsum(2, "ones")
_seg_4 = _make_segsum(4, None)
_seg_4_cnt = _make_segsum(4, "gather")


def _segsum(seg_fn, chunks, src, dst, sv128):
    args = list(chunks)
    if sv128 is not None:
        args.append(sv128)
    args += [src, dst.reshape(NTILES, NWIN, W),
             jnp.zeros((NPAD, 128), jnp.float32),
             jnp.ones((W, 128), jnp.float32)]
    return seg_fn(*args)


BLK = 1024
GRID = NPAD // BLK
NV = NPAD // 128


def _conv_body(nc, use_score, mask_kind, out_h):
    def body(*refs):
        i = pl.program_id(0)
        pos = 0
        aggs = refs[pos:pos + nc]; pos += nc
        cntA = refs[pos]; cntB = refs[pos + 1]; pos += 2
        xins = refs[pos:pos + nc]; pos += nc
        wrel = refs[pos]; brel = refs[pos + 1]; wroot = refs[pos + 2]
        pos += 3
        wsc = refs[pos]; pos += 1
        kept = None
        if mask_kind == "kept":
            kept = refs[pos]; pos += 1
        outs = list(refs[pos:])
        o = 0
        h_out = None
        if out_h:
            h_out = outs[o:o + 4]; o += 4
        cs_ref = outs[o]; o += 1
        sdot_ref = outs[o] if use_score else None

        inv = 1.0 / jnp.maximum(cntA[...][:, 0:1] + cntB[...][:, 0:1], 1.0)
        acc = jnp.zeros((BLK, H), jnp.float32)
        for c in range(nc):
            acc += (aggs[c][...] * inv) @ wrel[pl.ds(c * 128, 128), :]
            acc += xins[c][...] @ wroot[pl.ds(c * 128, 128), :]
        hv = jnp.maximum(acc + brel[...], 0.0)
        if out_h:
            for c in range(4):
                h_out[c][...] = hv[:, c * 128:(c + 1) * 128]
        nvb = BLK // 128
        if mask_kind == "kept":
            m3 = kept[...][:, :, None]
            hm = jnp.reshape(jnp.reshape(hv, (nvb, 128, H)) * m3, (BLK, H))
        else:
            rows = jax.lax.broadcasted_iota(jnp.int32, (BLK, 1), 0) + i * BLK
            hm = hv * (rows < N).astype(jnp.float32)
        cs = jnp.sum(hm, axis=0, keepdims=True)

        @pl.when(i == 0)
        def _():
            cs_ref[...] = jnp.zeros_like(cs_ref)

        cs_ref[...] += cs
        if use_score:
            sdot_ref[...] = jnp.sum(
                jnp.reshape(hv, (nvb, 128, H)) * wsc[...][None], axis=2)

    return body


def _conv_tc(aggs, cnts, xins, Wrel, brel, Wroot, wsc, kept, use_score,
             mask_kind, out_h):
    nc = len(aggs)
    chunk_spec = pl.BlockSpec((BLK, 128), lambda i: (i, 0))
    col_spec = pl.BlockSpec((BLK, 1), lambda i: (i, 0))
    full = lambda a: pl.BlockSpec(a.shape, lambda i: (0, 0))
    in_specs = ([chunk_spec] * nc + [chunk_spec, chunk_spec]
                + [chunk_spec] * nc
                + [full(Wrel), full(brel), full(Wroot), full(wsc)])
    args = list(aggs) + list(cnts) + list(xins) + [Wrel, brel, Wroot, wsc]
    if mask_kind == "kept":
        in_specs.append(pl.BlockSpec((BLK // 128, 128), lambda i: (i, 0)))
        args.append(kept)
    out_shape = []
    out_specs = []
    if out_h:
        out_shape += [jax.ShapeDtypeStruct((NPAD, 128), jnp.float32)] * 4
        out_specs += [chunk_spec] * 4
    out_shape.append(jax.ShapeDtypeStruct((1, H), jnp.float32))
    out_specs.append(pl.BlockSpec((1, H), lambda i: (0, 0)))
    if use_score:
        out_shape.append(jax.ShapeDtypeStruct((NPAD // 128, 128),
                                              jnp.float32))
        out_specs.append(pl.BlockSpec((BLK // 128, 128), lambda i: (i, 0)))
    return pl.pallas_call(
        _conv_body(nc, use_score, mask_kind, out_h),
        grid=(GRID,),
        in_specs=in_specs,
        out_specs=out_specs,
        out_shape=out_shape,
    )(*args)


NV = NPAD // 128


def _lane_shift_scan(x):
    sh = 1
    while sh < x.shape[1]:
        x = x + jnp.concatenate(
            [jnp.zeros((x.shape[0], sh), x.dtype), x[:, :-sh]], axis=1)
        sh *= 2
    return x


def _sub_shift_scan(x):
    sh = 1
    while sh < x.shape[0]:
        x = x + jnp.concatenate(
            [jnp.zeros((sh, x.shape[1]), x.dtype), x[:-sh]], axis=0)
        sh *= 2
    return x


def _topk_scale_body(sdot_ref, wsc_ref, h0, h1, h2, h3,
                     g0, g1, g2, g3, sv_ref, kept_out,
                     keptv, scalev):
    i = pl.program_id(0)

    @pl.when(i == 0)
    def _():
        w = wsc_ref[...]
        rin = jax.lax.rsqrt(jnp.sum(w * w))
        score = jnp.tanh(sdot_ref[...] * rin)
        flat = (jax.lax.broadcasted_iota(jnp.int32, (NV, 128), 0) * 128
                + jax.lax.broadcasted_iota(jnp.int32, (NV, 128), 1))
        valid = flat < N
        bits = jax.lax.bitcast_convert_type(score, jnp.int32)
        minint = jnp.int32(-2147483648)
        u = jnp.where(bits < 0, ~bits, bits | minint)
        svals = jnp.where(valid, u ^ minint, minint)

        def bsearch(j, tu):
            cand = tu | jax.lax.shift_left(jnp.int32(1), 31 - j)
            cnt = jnp.sum(jnp.where(svals >= (cand ^ minint), 1, 0))
            return jnp.where(cnt >= K, cand, tu)

        tu = jax.lax.fori_loop(0, 32, bsearch, jnp.int32(0))
        ts = tu ^ minint
        n_gt = jnp.sum(jnp.where(svals > ts, 1, 0))
        m = K - n_gt
        ties = (svals == ts).astype(jnp.int32)
        rs = _lane_shift_scan(ties)
        rowtot = rs[:, 127:128]
        pr = _sub_shift_scan(rowtot) - rowtot
        rank = rs - ties + pr
        keptb = (svals > ts) | ((ties > 0) & (rank < m))
        kf = keptb.astype(jnp.float32)
        keptv[...] = kf
        scalev[...] = score * kf
        kept_out[...] = kf

    nvb = BLK // 128
    sc3 = scalev[pl.ds(i * nvb, nvb), :][:, :, None]
    kc3 = keptv[pl.ds(i * nvb, nvb), :][:, :, None]
    for hin, gout in ((h0, g0), (h1, g1), (h2, g2), (h3, g3)):
        gout[...] = jnp.reshape(
            jnp.reshape(hin[...], (nvb, 128, 128)) * sc3, (BLK, 128))
    lane = jax.lax.broadcasted_iota(jnp.int32, (nvb, 128, 128), 2)
    sv_ref[...] = jnp.reshape(jnp.where(lane == 0, kc3, 0.0), (BLK, 128))


def _topk_scale_tc(sdot, wsc, hcs):
    chunk_spec = pl.BlockSpec((BLK, 128), lambda i: (i, 0))
    full = lambda a: pl.BlockSpec(a.shape, lambda i: (0, 0))
    return pl.pallas_call(
        _topk_scale_body,
        grid=(GRID,),
        in_specs=[full(sdot), full(wsc)] + [chunk_spec] * 4,
        out_specs=[chunk_spec] * 5 + [pl.BlockSpec((NV, 128),
                                                   lambda i: (0, 0))],
        out_shape=[jax.ShapeDtypeStruct((NPAD, 128), jnp.float32)] * 5
        + [jax.ShapeDtypeStruct((NV, 128), jnp.float32)],
        scratch_shapes=[pltpu.VMEM((NV, 128), jnp.float32)] * 2,
    )(sdot, wsc, *hcs)


def _head_body(cs1, cs2, cs3, w1, b1, w2, b2, o_ref):
    z1 = cs1[...] * (1.0 / N)
    z2 = cs2[...] * (1.0 / N)
    z3 = cs3[...] * (1.0 / K)
    zh = (z1 @ w1[pl.ds(0, H), :] + z2 @ w1[pl.ds(H, H), :]
          + z3 @ w1[pl.ds(2 * H, H), :])
    hh = jnp.maximum(zh + b1[...], 0.0)
    logits = hh @ w2[...] + b2[...]
    mx = jnp.max(logits, axis=-1, keepdims=True)
    lse = jnp.log(jnp.sum(jnp.exp(logits - mx), axis=-1, keepdims=True)) + mx
    o_ref[...] = logits - lse


def _head_tc(cs1, cs2, cs3, w1, b1, w2, b2):
    return pl.pallas_call(
        _head_body,
        out_shape=jax.ShapeDtypeStruct((1, w2.shape[1]), jnp.float32),
    )(cs1, cs2, cs3, w1, b1, w2, b2)


def kernel(x, edge_index, batch, conv1_Wrel, conv1_brel, conv1_Wroot, conv2_Wrel, conv2_brel, conv2_Wroot, conv3_Wrel, conv3_brel, conv3_Wroot, pool1_w, pool2_w, lin1_W, lin1_b, lin2_W, lin2_b):
    src = edge_index[0].astype(jnp.int32)
    dst = edge_index[1].astype(jnp.int32)

    xp = jnp.pad(x, ((0, NPAD - N), (0, 0)))
    xc = [xp[:, 0:128], xp[:, 128:256]]
    b1 = conv1_brel.reshape(1, H)
    b2 = conv2_brel.reshape(1, H)
    b3 = conv3_brel.reshape(1, H)
    wsc = pool1_w.reshape(1, H)
    lb1 = lin1_b.reshape(1, H)
    lb2 = lin2_b.reshape(1, lin2_W.shape[1])

    o = _segsum(_seg_2_cnt, xc, src, dst, None)
    agg1 = o[:2]
    cnt12 = o[2:4]
    h1_0, h1_1, h1_2, h1_3, cs1 = _conv_tc(
        agg1, cnt12, xc, conv1_Wrel, b1, conv1_Wroot, wsc, None,
        use_score=False, mask_kind="iota", out_h=True)
    h1 = [h1_0, h1_1, h1_2, h1_3]

    agg2 = _segsum(_seg_4, h1, src, dst, None)
    h2_0, h2_1, h2_2, h2_3, cs2, sdot = _conv_tc(
        agg2, cnt12, h1, conv2_Wrel, b2, conv2_Wroot, wsc, None,
        use_score=True, mask_kind="iota", out_h=True)
    h2 = [h2_0, h2_1, h2_2, h2_3]

    g0, g1, g2, g3, sv128, kept = _topk_scale_tc(sdot, wsc, h2)

    o = _segsum(_seg_4_cnt, [g0, g1, g2, g3], src, dst, sv128)
    agg3 = o[:4]
    cnt3 = o[4:6]
    (cs3,) = _conv_tc(
        agg3, cnt3, [g0, g1, g2, g3], conv3_Wrel, b3, conv3_Wroot, wsc, kept,
        use_score=False, mask_kind="kept", out_h=False)

    return _head_tc(cs1, cs2, cs3, lin1_W, lb1, lin2_W, lb2)
